# baseline (speedup 1.0000x reference)
import functools

import jax
import jax.numpy as jnp
from jax import lax
from jax.experimental import pallas as pl
from jax.experimental.pallas import tpu as pltpu
from jax.experimental.pallas import tpu_sc as plsc

N = 10000
E = 320000
NF = 128
H = 64
ED = 16
LD = 32

NUM_CORES = 2
NUM_SUBCORES = 16
NUM_TILES = NUM_CORES * NUM_SUBCORES

CHUNK = 128
SROWS = 6
SUPER = SROWS * CHUNK
NROW = E // CHUNK
ROWS_PER_TILE = 78
NSUP = 13
EXTRA = NROW - NUM_TILES * ROWS_PER_TILE
N_PAD = 10016
ZROWS = N_PAD // NUM_SUBCORES
ZHALF = ZROWS // 2
EBLK = 1280
PAIRS = E // 2
PROW = PAIRS // CHUNK
PPT = 39
PEXTRA = PROW - NUM_TILES * PPT
PBLK = EBLK // 2

_f32 = jnp.float32


def _tile_range(wid):
  row0 = ROWS_PER_TILE * wid + jnp.minimum(wid, EXTRA)
  return row0, row0 * CHUNK


def _gather_body(table, idx_hbm, out_hbm,
                 idx0, idx1, rows0, rows1, tidx, trows,
                 g0, g1, s0, s1, tsem):
  cid = lax.axis_index("c")
  sid = lax.axis_index("s")
  wid = sid * NUM_CORES + cid
  row0, e0 = _tile_range(wid)
  idx_b = [idx0, idx1]
  row_b = [rows0, rows1]
  gsem = [g0, g1]
  ssem = [s0, s1]
  gd = [None, None]
  sd = [None, None]

  def fire(s):
    b = s & 1
    pltpu.sync_copy(idx_hbm.at[pl.ds(row0 + SROWS * s, SROWS)], idx_b[b])
    gd[b] = [
        pltpu.async_copy(table.at[idx_b[b].at[j]],
                         row_b[b].at[pl.ds(j * CHUNK, CHUNK)], gsem[b])
        for j in range(SROWS)
    ]

  fire(0)
  fire(1)
  for s in range(NSUP):
    b = s & 1
    for d in gd[b]:
      d.wait()
    sd[b] = pltpu.async_copy(row_b[b],
                             out_hbm.at[pl.ds(e0 + SUPER * s, SUPER)],
                             ssem[b])
    if s + 2 < NSUP:
      sd[b].wait()
      sd[b] = None
      fire(s + 2)
  for b in range(2):
    if sd[b] is not None:
      sd[b].wait()

  @pl.when(wid < EXTRA)
  def _():
    pltpu.sync_copy(idx_hbm.at[pl.ds(row0 + ROWS_PER_TILE, 1)], tidx)
    pltpu.async_copy(table.at[tidx.at[0]], trows, tsem).wait()
    pltpu.sync_copy(trows,
                    out_hbm.at[pl.ds(e0 + ROWS_PER_TILE * CHUNK, CHUNK)])


def _make_gather(d):
  mesh = plsc.VectorSubcoreMesh(core_axis_name="c", subcore_axis_name="s",
                                num_cores=NUM_CORES,
                                num_subcores=NUM_SUBCORES)
  return pl.kernel(
      _gather_body,
      out_type=jax.ShapeDtypeStruct((E, d), _f32),
      mesh=mesh,
      compiler_params=pltpu.CompilerParams(use_tc_tiling_on_sc=False),
      scratch_types=[
          pltpu.VMEM((SROWS, CHUNK), jnp.int32),
          pltpu.VMEM((SROWS, CHUNK), jnp.int32),
          pltpu.VMEM((SUPER, d), _f32),
          pltpu.VMEM((SUPER, d), _f32),
          pltpu.VMEM((1, CHUNK), jnp.int32),
          pltpu.VMEM((CHUNK, d), _f32),
          pltpu.SemaphoreType.DMA,
          pltpu.SemaphoreType.DMA,
          pltpu.SemaphoreType.DMA,
          pltpu.SemaphoreType.DMA,
          pltpu.SemaphoreType.DMA,
      ],
  )


ZNROW = 4 * PROW
ZRPT = 156
ZEXTRA = ZNROW - NUM_TILES * ZRPT
ZNSUP = ZRPT // SROWS


def _zgather_body(table, idx_hbm, out_hbm,
                  idx0, idx1, rows0, rows1, tidx, trows,
                  g0, g1, s0, s1, tsem):
  cid = lax.axis_index("c")
  sid = lax.axis_index("s")
  wid = sid * NUM_CORES + cid
  row0 = ZRPT * wid + jnp.minimum(wid, ZEXTRA)
  e0 = row0 * CHUNK
  idx_b = [idx0, idx1]
  row_b = [rows0, rows1]
  gsem = [g0, g1]
  ssem = [s0, s1]
  gd = [None, None]
  sd = [None, None]

  def fire(s):
    b = s & 1
    pltpu.sync_copy(idx_hbm.at[pl.ds(row0 + SROWS * s, SROWS)], idx_b[b])
    gd[b] = [
        pltpu.async_copy(table.at[idx_b[b].at[j]],
                         row_b[b].at[pl.ds(j * CHUNK, CHUNK)], gsem[b])
        for j in range(SROWS)
    ]

  fire(0)
  fire(1)
  for s in range(ZNSUP):
    b = s & 1
    for d in gd[b]:
      d.wait()
    sd[b] = pltpu.async_copy(row_b[b],
                             out_hbm.at[pl.ds(e0 + SUPER * s, SUPER)],
                             ssem[b])
    if s + 2 < ZNSUP:
      sd[b].wait()
      sd[b] = None
      fire(s + 2)
  for b in range(2):
    if sd[b] is not None:
      sd[b].wait()

  @pl.when(wid < ZEXTRA)
  def _():
    pltpu.sync_copy(idx_hbm.at[pl.ds(row0 + ZRPT, 1)], tidx)
    pltpu.async_copy(table.at[tidx.at[0]], trows, tsem).wait()
    pltpu.sync_copy(trows, out_hbm.at[pl.ds(e0 + ZRPT * CHUNK, CHUNK)])


def _make_zgather():
  mesh = plsc.VectorSubcoreMesh(core_axis_name="c", subcore_axis_name="s",
                                num_cores=NUM_CORES,
                                num_subcores=NUM_SUBCORES)
  return pl.kernel(
      _zgather_body,
      out_type=jax.ShapeDtypeStruct((2 * E, LD), _f32),
      mesh=mesh,
      compiler_params=pltpu.CompilerParams(use_tc_tiling_on_sc=False),
      scratch_types=[
          pltpu.VMEM((SROWS, CHUNK), jnp.int32),
          pltpu.VMEM((SROWS, CHUNK), jnp.int32),
          pltpu.VMEM((SUPER, LD), _f32),
          pltpu.VMEM((SUPER, LD), _f32),
          pltpu.VMEM((1, CHUNK), jnp.int32),
          pltpu.VMEM((CHUNK, LD), _f32),
          pltpu.SemaphoreType.DMA,
          pltpu.SemaphoreType.DMA,
          pltpu.SemaphoreType.DMA,
          pltpu.SemaphoreType.DMA,
          pltpu.SemaphoreType.DMA,
      ],
  )


def _conv_body(e2_hbm, src_hbm, dst_hbm, h_hbm, zeros_hbm, out_hbm,
               slo0, slo1, slo2, shi0, shi1, shi2,
               dlo0, dlo1, dlo2, dhi0, dhi1, dhi2,
               eb0, eb1, hlo0, hlo1, hlo2, hhi0, hhi1, hhi2, acc_sh,
               ge0, ge1, gl0, gl1, gl2, gh0, gh1, gh2,
               as0, as1, as2, tsem):
  cid = lax.axis_index("c")
  sid = lax.axis_index("s")
  wid = cid * NUM_SUBCORES + sid
  p0 = PPT * wid + jnp.minimum(wid, PEXTRA)

  pltpu.sync_copy(zeros_hbm, hlo0.at[pl.ds(0, CHUNK)])
  for zz in range(5):
    rows = CHUNK if zz < 4 else ZROWS - 4 * CHUNK
    pltpu.sync_copy(hlo0.at[pl.ds(0, rows)],
                    acc_sh.at[pl.ds(sid * ZROWS + zz * CHUNK, rows)])
  plsc.subcore_barrier()

  slo = [slo0, slo1, slo2]
  shi = [shi0, shi1, shi2]
  dlo = [dlo0, dlo1, dlo2]
  dhi = [dhi0, dhi1, dhi2]
  ebuf = [eb0, eb1]
  hlo = [hlo0, hlo1, hlo2]
  hhi = [hhi0, hhi1, hhi2]
  gesem = [ge0, ge1]
  glsem = [gl0, gl1, gl2]
  ghsem = [gh0, gh1, gh2]
  asem = [as0, as1, as2]
  ed = [None, None]
  gld = [None, None, None]
  ghd = [None, None, None]
  ad = [None, None, None]

  def fire(s):
    b2 = s & 1
    b3 = s % 3
    p = p0 + s
    pltpu.sync_copy(src_hbm.at[pl.ds(p, 1)], slo[b3])
    pltpu.sync_copy(src_hbm.at[pl.ds(PROW + p, 1)], shi[b3])
    pltpu.sync_copy(dst_hbm.at[pl.ds(p, 1)], dlo[b3])
    pltpu.sync_copy(dst_hbm.at[pl.ds(PROW + p, 1)], dhi[b3])
    ed[b2] = pltpu.async_copy(e2_hbm.at[pl.ds(p * CHUNK, CHUNK)], ebuf[b2],
                              gesem[b2])
    gld[b3] = pltpu.async_copy(h_hbm.at[slo[b3].at[0]], hlo[b3], glsem[b3])
    ghd[b3] = pltpu.async_copy(h_hbm.at[shi[b3].at[0]], hhi[b3], ghsem[b3])

  def step(s):
    b2 = s & 1
    b3 = s % 3
    ed[b2].wait()
    gld[b3].wait()
    ghd[b3].wait()

    @plsc.parallel_loop(0, CHUNK, step=1, unroll=1)
    def body(r):
      for c in range(4):
        lo = jnp.maximum(hlo[b3][r, pl.ds(c * 16, 16)]
                         + ebuf[b2][r, pl.ds(c * 16, 16)], 0.0)
        hlo[b3][r, pl.ds(c * 16, 16)] = lo
        hi = jnp.maximum(hhi[b3][r, pl.ds(c * 16, 16)]
                         + ebuf[b2][r, pl.ds(64 + c * 16, 16)], 0.0)
        hhi[b3][r, pl.ds(c * 16, 16)] = hi

    ad[b3] = [
        pltpu.async_copy(hlo[b3], acc_sh.at[dlo[b3].at[0]], asem[b3],
                         add=True),
        pltpu.async_copy(hhi[b3], acc_sh.at[dhi[b3].at[0]], asem[b3],
                         add=True),
    ]

  def drain(s):
    if s < 0:
      return
    b3 = s % 3
    if ad[b3] is not None:
      for d in ad[b3]:
        d.wait()
      ad[b3] = None

  fire(0)
  fire(1)
  for s in range(PPT):
    step(s)
    if s + 2 < PPT:
      drain(s - 1)
      fire(s + 2)
  drain(PPT - 3)
  drain(PPT - 2)
  drain(PPT - 1)

  @pl.when(wid < PEXTRA)
  def _():
    fire(PPT)
    step(PPT)
    drain(PPT)

  plsc.subcore_barrier()
  for hlf in range(2):
    pltpu.sync_copy(acc_sh.at[pl.ds(sid * ZROWS + hlf * ZHALF, ZHALF)],
                    hlo0.at[pl.ds(0, ZHALF)])
    pltpu.sync_copy(hlo0.at[pl.ds(0, ZHALF)],
                    out_hbm.at[cid, pl.ds(sid * ZROWS + hlf * ZHALF, ZHALF)])


def _make_conv():
  mesh = plsc.VectorSubcoreMesh(core_axis_name="c", subcore_axis_name="s",
                                num_cores=NUM_CORES,
                                num_subcores=NUM_SUBCORES)
  return pl.kernel(
      _conv_body,
      out_type=jax.ShapeDtypeStruct((NUM_CORES, N_PAD, H), _f32),
      mesh=mesh,
      compiler_params=pltpu.CompilerParams(use_tc_tiling_on_sc=False),
      scratch_types=(
          [pltpu.VMEM((1, CHUNK), jnp.int32)] * 12
          + [pltpu.VMEM((CHUNK, 2 * H), _f32)] * 2
          + [pltpu.VMEM((CHUNK, H), _f32)] * 6
          + [pltpu.VMEM_SHARED((N_PAD, H), _f32)]
          + [pltpu.SemaphoreType.DMA] * 12
      ),
  )


def _make_scatter():
  mesh = plsc.VectorSubcoreMesh(core_axis_name="c", subcore_axis_name="s",
                                num_cores=NUM_CORES,
                                num_subcores=NUM_SUBCORES)
  return pl.kernel(
      _scatter_body,
      out_type=jax.ShapeDtypeStruct((NUM_CORES, N_PAD, H), _f32),
      mesh=mesh,
      compiler_params=pltpu.CompilerParams(use_tc_tiling_on_sc=False),
      scratch_types=[
          pltpu.VMEM((SROWS_SC, CHUNK), jnp.int32),
          pltpu.VMEM((SROWS_SC, CHUNK), jnp.int32),
          pltpu.VMEM((SUPER_SC, H), _f32),
          pltpu.VMEM((SUPER_SC, H), _f32),
          pltpu.VMEM((1, CHUNK), jnp.int32),
          pltpu.VMEM((CHUNK, H), _f32),
          pltpu.VMEM_SHARED((N_PAD, H), _f32),
          pltpu.SemaphoreType.DMA,
          pltpu.SemaphoreType.DMA,
          pltpu.SemaphoreType.DMA,
          pltpu.SemaphoreType.DMA,
          pltpu.SemaphoreType.DMA,
      ],
  )


def _dgT(lhs, rhs):
  return lax.dot_general(lhs, rhs, (((0,), (0,)), ((), ())),
                         preferred_element_type=_f32)


def _node_emb_body(x_ref, w_ref, b_ref, o_ref):
  o_ref[...] = jnp.dot(x_ref[...], w_ref[...],
                       preferred_element_type=_f32) + b_ref[...]


def _node_emb(x, w, b):
  return pl.pallas_call(
      _node_emb_body,
      out_shape=jax.ShapeDtypeStruct((N, H), _f32),
  )(x, w, b.reshape(1, H))


def _edge_lin_body(eat_lo_ref, eat_hi_ref, a1_ref, c1_ref, a2_ref, c2_ref,
                   o1_ref, o2_ref):
  lo = eat_lo_ref[...]
  hi = eat_hi_ref[...]
  o1_ref[...] = jnp.concatenate(
      [_dgT(lo, a1_ref[...]) + c1_ref[...],
       _dgT(hi, a1_ref[...]) + c1_ref[...]], axis=1)
  o2_ref[...] = jnp.concatenate(
      [_dgT(lo, a2_ref[...]) + c2_ref[...],
       _dgT(hi, a2_ref[...]) + c2_ref[...]], axis=1)


def _edge_lin(ea_t, a1, c1, a2, c2):
  nblk = PAIRS // PBLK
  out = jax.ShapeDtypeStruct((PAIRS, 2 * H), _f32)
  return pl.pallas_call(
      _edge_lin_body,
      grid=(nblk,),
      in_specs=[
          pl.BlockSpec((ED, PBLK), lambda i: (0, i)),
          pl.BlockSpec((ED, PBLK), lambda i: (0, i + PAIRS // PBLK)),
          pl.BlockSpec((ED, H), lambda i: (0, 0)),
          pl.BlockSpec((1, H), lambda i: (0, 0)),
          pl.BlockSpec((ED, H), lambda i: (0, 0)),
          pl.BlockSpec((1, H), lambda i: (0, 0)),
      ],
      out_specs=[pl.BlockSpec((PBLK, 2 * H), lambda i: (i, 0)),
                 pl.BlockSpec((PBLK, 2 * H), lambda i: (i, 0))],
      out_shape=[out, out],
  )(ea_t, ea_t, a1, c1.reshape(1, H), a2, c2.reshape(1, H))


def _node_update_body(emit_z, h_ref, acc_ref, scale_ref, w1_ref, b1_ref,
                      w2_ref, b2_ref, g_ref, be_ref, ew_ref, eb_ref, o_ref):
  aggr = acc_ref[0, :N, :] + acc_ref[1, :N, :]
  h = h_ref[...]
  t = scale_ref[0, 0] * h + aggr
  t = jnp.maximum(jnp.dot(t, w1_ref[...],
                          preferred_element_type=_f32) + b1_ref[...], 0.0)
  t = jnp.dot(t, w2_ref[...], preferred_element_type=_f32) + b2_ref[...]
  mean = jnp.mean(t, axis=0, keepdims=True)
  var = jnp.mean((t - mean) ** 2, axis=0, keepdims=True)
  t = (t - mean) * lax.rsqrt(var + 1e-5) * g_ref[...] + be_ref[...]
  hn = (h + jnp.maximum(t, 0.0)) * 0.5
  if emit_z:
    o_ref[...] = jnp.dot(hn, ew_ref[...],
                         preferred_element_type=_f32) + eb_ref[...]
  else:
    o_ref[...] = hn


def _node_update(h, acc, scale, w1, b1, w2, b2, gamma, beta, enc_w, enc_b,
                 emit_z):
  out_d = LD if emit_z else H
  return pl.pallas_call(
      functools.partial(_node_update_body, emit_z),
      out_shape=jax.ShapeDtypeStruct((N, out_d), _f32),
  )(h, acc, scale.reshape(1, 1), w1, b1.reshape(1, H), w2, b2.reshape(1, H),
    gamma.reshape(1, H), beta.reshape(1, H), enc_w, enc_b.reshape(1, LD))


def _head_body(zz_ref, eatl_ref, eath_ref, d1l_ref, d1h_ref, db1_ref,
               d2_ref, db2_ref, m1l_ref, m1h_ref, m1c_ref, m1d_ref, mb1_ref,
               m2_ref, mb2_ref, m3_ref, mb3_ref, ol_ref, oh_ref):
  zz = zz_ref[...]
  dot = lambda a, b: jnp.dot(a, b, preferred_element_type=_f32)

  def group(d1_ref, m1_ref, eat_ref, o_ref):
    eat = eat_ref[...]
    t = jnp.maximum(dot(zz, d1_ref[...]) + db1_ref[...], 0.0)
    rec_t = lax.dot_general(d2_ref[...], t, (((0,), (1,)), ((), ())),
                            preferred_element_type=_f32) + db2_ref[...]
    diff = rec_t - eat
    err = jnp.mean(diff * diff, axis=0, keepdims=True)
    m = jnp.maximum(dot(zz, m1_ref[...])
                    + _dgT(eat, m1c_ref[...])
                    + _dgT(err, m1d_ref[...])
                    + mb1_ref[...], 0.0)
    m = jnp.maximum(dot(m, m2_ref[...]) + mb2_ref[...], 0.0)
    o_ref[...] = lax.dot_general(m3_ref[...], m, (((0,), (1,)), ((), ())),
                                 preferred_element_type=_f32) + mb3_ref[...]

  group(d1l_ref, m1l_ref, eatl_ref, ol_ref)
  group(d1h_ref, m1h_ref, eath_ref, oh_ref)


def _head(zz, ea_t, d1, db1, d2, db2, m1, mb1, m2, mb2, m3, mb3):
  nblk = PAIRS // PBLK
  full = lambda shape: pl.BlockSpec(shape, lambda i: tuple(0 for _ in shape))
  zeros64 = jnp.zeros((H, d1.shape[1]), _f32)
  d1_lo = jnp.concatenate([d1, zeros64], axis=0)
  d1_hi = jnp.concatenate([zeros64, d1], axis=0)
  m1ab = m1[:2 * LD]
  z50 = jnp.zeros((H, 50), _f32)
  m1_lo = jnp.concatenate([m1ab, z50], axis=0)
  m1_hi = jnp.concatenate([z50, m1ab], axis=0)
  out = jax.ShapeDtypeStruct((2, PAIRS), _f32)
  return pl.pallas_call(
      _head_body,
      grid=(nblk,),
      in_specs=[
          pl.BlockSpec((PBLK, 2 * H), lambda i: (i, 0)),
          pl.BlockSpec((ED, PBLK), lambda i: (0, i)),
          pl.BlockSpec((ED, PBLK), lambda i: (0, i + PAIRS // PBLK)),
          full((2 * H, H)), full((2 * H, H)), full((1, H)),
          full((H, ED)), full((ED, 1)),
          full((2 * H, 50)), full((2 * H, 50)), full((ED, 50)),
          full((1, 50)), full((1, 50)),
          full((50, 25)), full((1, 25)),
          full((25, 2)), full((2, 1)),
      ],
      out_specs=[pl.BlockSpec((2, PBLK), lambda i: (0, i)),
                 pl.BlockSpec((2, PBLK), lambda i: (0, i))],
      out_shape=[out, out],
  )(zz, ea_t, ea_t,
    d1_lo, d1_hi, db1.reshape(1, H),
    d2, db2.reshape(ED, 1),
    m1_lo, m1_hi, m1[2 * LD:2 * LD + ED],
    m1[2 * LD + ED:].reshape(1, 50), mb1.reshape(1, 50),
    m2, mb2.reshape(1, 25),
    m3, mb3.reshape(2, 1))


@jax.jit
def _run(x, edge_index, edge_attr, params):
  src2d = edge_index[0].reshape(NROW, CHUNK)
  dst2d = edge_index[1].reshape(NROW, CHUNK)
  ea_t = edge_attr.T
  zeros_z = jnp.zeros((CHUNK, H), _f32)

  we, be = params['edge_emb']
  folded = []
  for conv in params['convs']:
    wl, bl = conv['lin_edge']
    folded.append((we @ wl, be @ wl + bl))

  zgather = _make_zgather()
  conv_sc = _make_conv()

  (a1, c1), (a2, c2) = folded
  e2_1, e2_2 = _edge_lin(ea_t, a1, c1, a2, c2)
  e2s = [e2_1, e2_2]

  h = _node_emb(x, params['node_emb'][0], params['node_emb'][1])

  for li, conv in enumerate(params['convs']):
    acc = conv_sc(e2s[li], src2d, dst2d, h, zeros_z)
    scale = (1.0 + conv['eps']).astype(_f32)
    emit_z = li == len(params['convs']) - 1
    h = _node_update(h, acc, scale,
                     conv['nn1'][0], conv['nn1'][1],
                     conv['nn2'][0], conv['nn2'][1],
                     conv['bn_gamma'], conv['bn_beta'],
                     params['enc'][0], params['enc'][1], emit_z)

  z = h
  src_i = edge_index[0]
  dst_i = edge_index[1]
  zidx = jnp.stack([src_i[:PAIRS], dst_i[:PAIRS],
                    src_i[PAIRS:], dst_i[PAIRS:]], axis=1).reshape(-1)
  zz = zgather(z, zidx.reshape(ZNROW, CHUNK)).reshape(PAIRS, 4 * LD)
  out_lo, out_hi = _head(zz, ea_t,
                         params['dec1'][0], params['dec1'][1],
                         params['dec2'][0], params['dec2'][1],
                         params['mlp1'][0], params['mlp1'][1],
                         params['mlp2'][0], params['mlp2'][1],
                         params['mlp3'][0], params['mlp3'][1])
  out_t = jnp.concatenate([out_lo, out_hi], axis=1)
  return out_t.T


def kernel(x, edge_index, edge_attr, params):
  return _run(x, edge_index, edge_attr, params)


# speedup vs baseline: 1.0577x; 1.0577x over previous
import functools

import jax
import jax.numpy as jnp
from jax import lax
from jax.experimental import pallas as pl
from jax.experimental.pallas import tpu as pltpu
from jax.experimental.pallas import tpu_sc as plsc

N = 10000
E = 320000
NF = 128
H = 64
ED = 16
LD = 32

NUM_CORES = 2
NUM_SUBCORES = 16
NUM_TILES = NUM_CORES * NUM_SUBCORES

CHUNK = 128
SROWS = 6
SUPER = SROWS * CHUNK
NROW = E // CHUNK
ROWS_PER_TILE = 78
NSUP = 13
EXTRA = NROW - NUM_TILES * ROWS_PER_TILE
N_PAD = 10016
ZROWS = N_PAD // NUM_SUBCORES
ZHALF = ZROWS // 2
EBLK = 1280
PAIRS = E // 2
PROW = PAIRS // CHUNK
PPT = 39
PEXTRA = PROW - NUM_TILES * PPT
PBLK = EBLK // 2

_f32 = jnp.float32


def _tile_range(wid):
  row0 = ROWS_PER_TILE * wid + jnp.minimum(wid, EXTRA)
  return row0, row0 * CHUNK


def _gather_body(table, idx_hbm, out_hbm,
                 idx0, idx1, rows0, rows1, tidx, trows,
                 g0, g1, s0, s1, tsem):
  cid = lax.axis_index("c")
  sid = lax.axis_index("s")
  wid = sid * NUM_CORES + cid
  row0, e0 = _tile_range(wid)
  idx_b = [idx0, idx1]
  row_b = [rows0, rows1]
  gsem = [g0, g1]
  ssem = [s0, s1]
  gd = [None, None]
  sd = [None, None]

  def fire(s):
    b = s & 1
    pltpu.sync_copy(idx_hbm.at[pl.ds(row0 + SROWS * s, SROWS)], idx_b[b])
    gd[b] = [
        pltpu.async_copy(table.at[idx_b[b].at[j]],
                         row_b[b].at[pl.ds(j * CHUNK, CHUNK)], gsem[b])
        for j in range(SROWS)
    ]

  fire(0)
  fire(1)
  for s in range(NSUP):
    b = s & 1
    for d in gd[b]:
      d.wait()
    sd[b] = pltpu.async_copy(row_b[b],
                             out_hbm.at[pl.ds(e0 + SUPER * s, SUPER)],
                             ssem[b])
    if s + 2 < NSUP:
      sd[b].wait()
      sd[b] = None
      fire(s + 2)
  for b in range(2):
    if sd[b] is not None:
      sd[b].wait()

  @pl.when(wid < EXTRA)
  def _():
    pltpu.sync_copy(idx_hbm.at[pl.ds(row0 + ROWS_PER_TILE, 1)], tidx)
    pltpu.async_copy(table.at[tidx.at[0]], trows, tsem).wait()
    pltpu.sync_copy(trows,
                    out_hbm.at[pl.ds(e0 + ROWS_PER_TILE * CHUNK, CHUNK)])


def _make_gather(d):
  mesh = plsc.VectorSubcoreMesh(core_axis_name="c", subcore_axis_name="s",
                                num_cores=NUM_CORES,
                                num_subcores=NUM_SUBCORES)
  return pl.kernel(
      _gather_body,
      out_type=jax.ShapeDtypeStruct((E, d), _f32),
      mesh=mesh,
      compiler_params=pltpu.CompilerParams(use_tc_tiling_on_sc=False),
      scratch_types=[
          pltpu.VMEM((SROWS, CHUNK), jnp.int32),
          pltpu.VMEM((SROWS, CHUNK), jnp.int32),
          pltpu.VMEM((SUPER, d), _f32),
          pltpu.VMEM((SUPER, d), _f32),
          pltpu.VMEM((1, CHUNK), jnp.int32),
          pltpu.VMEM((CHUNK, d), _f32),
          pltpu.SemaphoreType.DMA,
          pltpu.SemaphoreType.DMA,
          pltpu.SemaphoreType.DMA,
          pltpu.SemaphoreType.DMA,
          pltpu.SemaphoreType.DMA,
      ],
  )


ZNROW = 4 * PROW
ZRPT = 156
ZEXTRA = ZNROW - NUM_TILES * ZRPT
ZNSUP = ZRPT // SROWS


def _zgather_body(table, idx_hbm, out_hbm,
                  idx0, idx1, rows0, rows1, tidx, trows,
                  g0, g1, s0, s1, tsem):
  cid = lax.axis_index("c")
  sid = lax.axis_index("s")
  wid = sid * NUM_CORES + cid
  row0 = ZRPT * wid + jnp.minimum(wid, ZEXTRA)
  e0 = row0 * CHUNK
  idx_b = [idx0, idx1]
  row_b = [rows0, rows1]
  gsem = [g0, g1]
  ssem = [s0, s1]
  gd = [None, None]
  sd = [None, None]

  def fire(s):
    b = s & 1
    pltpu.sync_copy(idx_hbm.at[pl.ds(row0 + SROWS * s, SROWS)], idx_b[b])
    gd[b] = [
        pltpu.async_copy(table.at[idx_b[b].at[j]],
                         row_b[b].at[pl.ds(j * CHUNK, CHUNK)], gsem[b])
        for j in range(SROWS)
    ]

  fire(0)
  fire(1)
  for s in range(ZNSUP):
    b = s & 1
    for d in gd[b]:
      d.wait()
    sd[b] = pltpu.async_copy(row_b[b],
                             out_hbm.at[pl.ds(e0 + SUPER * s, SUPER)],
                             ssem[b])
    if s + 2 < ZNSUP:
      sd[b].wait()
      sd[b] = None
      fire(s + 2)
  for b in range(2):
    if sd[b] is not None:
      sd[b].wait()

  @pl.when(wid < ZEXTRA)
  def _():
    pltpu.sync_copy(idx_hbm.at[pl.ds(row0 + ZRPT, 1)], tidx)
    pltpu.async_copy(table.at[tidx.at[0]], trows, tsem).wait()
    pltpu.sync_copy(trows, out_hbm.at[pl.ds(e0 + ZRPT * CHUNK, CHUNK)])


def _make_zgather():
  mesh = plsc.VectorSubcoreMesh(core_axis_name="c", subcore_axis_name="s",
                                num_cores=NUM_CORES,
                                num_subcores=NUM_SUBCORES)
  return pl.kernel(
      _zgather_body,
      out_type=jax.ShapeDtypeStruct((2 * E, LD), _f32),
      mesh=mesh,
      compiler_params=pltpu.CompilerParams(use_tc_tiling_on_sc=False),
      scratch_types=[
          pltpu.VMEM((SROWS, CHUNK), jnp.int32),
          pltpu.VMEM((SROWS, CHUNK), jnp.int32),
          pltpu.VMEM((SUPER, LD), _f32),
          pltpu.VMEM((SUPER, LD), _f32),
          pltpu.VMEM((1, CHUNK), jnp.int32),
          pltpu.VMEM((CHUNK, LD), _f32),
          pltpu.SemaphoreType.DMA,
          pltpu.SemaphoreType.DMA,
          pltpu.SemaphoreType.DMA,
          pltpu.SemaphoreType.DMA,
          pltpu.SemaphoreType.DMA,
      ],
  )


def _conv_body(e2_hbm, src_hbm, dst_hbm, h_hbm, zeros_hbm, out_hbm,
               slo0, slo1, slo2, shi0, shi1, shi2,
               dlo0, dlo1, dlo2, dhi0, dhi1, dhi2,
               eb0, eb1, hlo0, hlo1, hlo2, hhi0, hhi1, hhi2, acc_sh,
               ge0, ge1, gl0, gl1, gl2, gh0, gh1, gh2,
               as0, as1, as2, tsem):
  cid = lax.axis_index("c")
  sid = lax.axis_index("s")
  wid = cid * NUM_SUBCORES + sid
  p0 = PPT * wid + jnp.minimum(wid, PEXTRA)

  pltpu.sync_copy(zeros_hbm, hlo0.at[pl.ds(0, CHUNK)])
  for zz in range(5):
    rows = CHUNK if zz < 4 else ZROWS - 4 * CHUNK
    pltpu.sync_copy(hlo0.at[pl.ds(0, rows)],
                    acc_sh.at[pl.ds(sid * ZROWS + zz * CHUNK, rows)])
  plsc.subcore_barrier()

  slo = [slo0, slo1, slo2]
  shi = [shi0, shi1, shi2]
  dlo = [dlo0, dlo1, dlo2]
  dhi = [dhi0, dhi1, dhi2]
  ebuf = [eb0, eb1]
  hlo = [hlo0, hlo1, hlo2]
  hhi = [hhi0, hhi1, hhi2]
  gesem = [ge0, ge1]
  glsem = [gl0, gl1, gl2]
  ghsem = [gh0, gh1, gh2]
  asem = [as0, as1, as2]
  ed = [None, None]
  gld = [None, None, None]
  ghd = [None, None, None]
  ad = [None, None, None]

  def fire(s):
    b2 = s & 1
    b3 = s % 3
    p = p0 + s
    pltpu.sync_copy(src_hbm.at[pl.ds(p, 1)], slo[b3])
    pltpu.sync_copy(src_hbm.at[pl.ds(PROW + p, 1)], shi[b3])
    pltpu.sync_copy(dst_hbm.at[pl.ds(p, 1)], dlo[b3])
    pltpu.sync_copy(dst_hbm.at[pl.ds(PROW + p, 1)], dhi[b3])
    ed[b2] = pltpu.async_copy(e2_hbm.at[pl.ds(p * CHUNK, CHUNK)], ebuf[b2],
                              gesem[b2])
    gld[b3] = pltpu.async_copy(h_hbm.at[slo[b3].at[0]], hlo[b3], glsem[b3])
    ghd[b3] = pltpu.async_copy(h_hbm.at[shi[b3].at[0]], hhi[b3], ghsem[b3])

  def step(s):
    b2 = s & 1
    b3 = s % 3
    ed[b2].wait()
    gld[b3].wait()
    ghd[b3].wait()

    @plsc.parallel_loop(0, CHUNK, step=1, unroll=1)
    def body(r):
      for c in range(4):
        lo = jnp.maximum(hlo[b3][r, pl.ds(c * 16, 16)]
                         + ebuf[b2][r, pl.ds(c * 16, 16)], 0.0)
        hlo[b3][r, pl.ds(c * 16, 16)] = lo
        hi = jnp.maximum(hhi[b3][r, pl.ds(c * 16, 16)]
                         + ebuf[b2][r, pl.ds(64 + c * 16, 16)], 0.0)
        hhi[b3][r, pl.ds(c * 16, 16)] = hi

    ad[b3] = [
        pltpu.async_copy(hlo[b3], acc_sh.at[dlo[b3].at[0]], asem[b3],
                         add=True),
        pltpu.async_copy(hhi[b3], acc_sh.at[dhi[b3].at[0]], asem[b3],
                         add=True),
    ]

  def drain(s):
    if s < 0:
      return
    b3 = s % 3
    if ad[b3] is not None:
      for d in ad[b3]:
        d.wait()
      ad[b3] = None

  fire(0)
  fire(1)
  for s in range(PPT):
    step(s)
    if s + 2 < PPT:
      drain(s - 1)
      fire(s + 2)
  drain(PPT - 3)
  drain(PPT - 2)
  drain(PPT - 1)

  @pl.when(wid < PEXTRA)
  def _():
    fire(PPT)
    step(PPT)
    drain(PPT)

  plsc.subcore_barrier()
  for hlf in range(2):
    pltpu.sync_copy(acc_sh.at[pl.ds(sid * ZROWS + hlf * ZHALF, ZHALF)],
                    hlo0.at[pl.ds(0, ZHALF)])
    pltpu.sync_copy(hlo0.at[pl.ds(0, ZHALF)],
                    out_hbm.at[cid, pl.ds(sid * ZROWS + hlf * ZHALF, ZHALF)])


def _make_conv():
  mesh = plsc.VectorSubcoreMesh(core_axis_name="c", subcore_axis_name="s",
                                num_cores=NUM_CORES,
                                num_subcores=NUM_SUBCORES)
  return pl.kernel(
      _conv_body,
      out_type=jax.ShapeDtypeStruct((NUM_CORES, N_PAD, H), _f32),
      mesh=mesh,
      compiler_params=pltpu.CompilerParams(use_tc_tiling_on_sc=False),
      scratch_types=(
          [pltpu.VMEM((1, CHUNK), jnp.int32)] * 12
          + [pltpu.VMEM((CHUNK, 2 * H), _f32)] * 2
          + [pltpu.VMEM((CHUNK, H), _f32)] * 6
          + [pltpu.VMEM_SHARED((N_PAD, H), _f32)]
          + [pltpu.SemaphoreType.DMA] * 12
      ),
  )


def _make_scatter():
  mesh = plsc.VectorSubcoreMesh(core_axis_name="c", subcore_axis_name="s",
                                num_cores=NUM_CORES,
                                num_subcores=NUM_SUBCORES)
  return pl.kernel(
      _scatter_body,
      out_type=jax.ShapeDtypeStruct((NUM_CORES, N_PAD, H), _f32),
      mesh=mesh,
      compiler_params=pltpu.CompilerParams(use_tc_tiling_on_sc=False),
      scratch_types=[
          pltpu.VMEM((SROWS_SC, CHUNK), jnp.int32),
          pltpu.VMEM((SROWS_SC, CHUNK), jnp.int32),
          pltpu.VMEM((SUPER_SC, H), _f32),
          pltpu.VMEM((SUPER_SC, H), _f32),
          pltpu.VMEM((1, CHUNK), jnp.int32),
          pltpu.VMEM((CHUNK, H), _f32),
          pltpu.VMEM_SHARED((N_PAD, H), _f32),
          pltpu.SemaphoreType.DMA,
          pltpu.SemaphoreType.DMA,
          pltpu.SemaphoreType.DMA,
          pltpu.SemaphoreType.DMA,
          pltpu.SemaphoreType.DMA,
      ],
  )


def _dgT(lhs, rhs):
  return lax.dot_general(lhs, rhs, (((0,), (0,)), ((), ())),
                         preferred_element_type=_f32)


def _node_emb_body(x_ref, w_ref, b_ref, o_ref):
  o_ref[...] = jnp.dot(x_ref[...], w_ref[...],
                       preferred_element_type=_f32) + b_ref[...]


def _node_emb(x, w, b):
  return pl.pallas_call(
      _node_emb_body,
      out_shape=jax.ShapeDtypeStruct((N, H), _f32),
  )(x, w, b.reshape(1, H))


def _edge_lin_body(eat_lo_ref, eat_hi_ref, a1_ref, c1_ref, a2_ref, c2_ref,
                   o1_ref, o2_ref):
  lo = eat_lo_ref[...]
  hi = eat_hi_ref[...]
  o1_ref[...] = jnp.concatenate(
      [_dgT(lo, a1_ref[...]) + c1_ref[...],
       _dgT(hi, a1_ref[...]) + c1_ref[...]], axis=1)
  o2_ref[...] = jnp.concatenate(
      [_dgT(lo, a2_ref[...]) + c2_ref[...],
       _dgT(hi, a2_ref[...]) + c2_ref[...]], axis=1)


def _edge_lin(ea_t, a1, c1, a2, c2):
  eblk = 2 * PBLK
  nblk = PAIRS // eblk
  out = jax.ShapeDtypeStruct((PAIRS, 2 * H), _f32)
  return pl.pallas_call(
      _edge_lin_body,
      grid=(nblk,),
      in_specs=[
          pl.BlockSpec((ED, eblk), lambda i: (0, i)),
          pl.BlockSpec((ED, eblk), lambda i: (0, i + PAIRS // eblk)),
          pl.BlockSpec((ED, H), lambda i: (0, 0)),
          pl.BlockSpec((1, H), lambda i: (0, 0)),
          pl.BlockSpec((ED, H), lambda i: (0, 0)),
          pl.BlockSpec((1, H), lambda i: (0, 0)),
      ],
      out_specs=[pl.BlockSpec((eblk, 2 * H), lambda i: (i, 0)),
                 pl.BlockSpec((eblk, 2 * H), lambda i: (i, 0))],
      out_shape=[out, out],
  )(ea_t, ea_t, a1, c1.reshape(1, H), a2, c2.reshape(1, H))


def _node_update_body(emit_z, h_ref, acc_ref, scale_ref, w1_ref, b1_ref,
                      w2_ref, b2_ref, g_ref, be_ref, ew_ref, eb_ref, o_ref):
  aggr = acc_ref[0, :N, :] + acc_ref[1, :N, :]
  h = h_ref[...]
  t = scale_ref[0, 0] * h + aggr
  t = jnp.maximum(jnp.dot(t, w1_ref[...],
                          preferred_element_type=_f32) + b1_ref[...], 0.0)
  t = jnp.dot(t, w2_ref[...], preferred_element_type=_f32) + b2_ref[...]
  mean = jnp.mean(t, axis=0, keepdims=True)
  var = jnp.mean((t - mean) ** 2, axis=0, keepdims=True)
  t = (t - mean) * lax.rsqrt(var + 1e-5) * g_ref[...] + be_ref[...]
  hn = (h + jnp.maximum(t, 0.0)) * 0.5
  if emit_z:
    o_ref[...] = jnp.dot(hn, ew_ref[...],
                         preferred_element_type=_f32) + eb_ref[...]
  else:
    o_ref[...] = hn


def _node_update(h, acc, scale, w1, b1, w2, b2, gamma, beta, enc_w, enc_b,
                 emit_z):
  out_d = LD if emit_z else H
  return pl.pallas_call(
      functools.partial(_node_update_body, emit_z),
      out_shape=jax.ShapeDtypeStruct((N, out_d), _f32),
  )(h, acc, scale.reshape(1, 1), w1, b1.reshape(1, H), w2, b2.reshape(1, H),
    gamma.reshape(1, H), beta.reshape(1, H), enc_w, enc_b.reshape(1, LD))


def _head_body(zz_ref, eatl_ref, eath_ref, d1l_ref, d1h_ref, db1_ref,
               d2_ref, db2_ref, m1l_ref, m1h_ref, m1c_ref, m1d_ref, mb1_ref,
               m2_ref, mb2_ref, m3_ref, mb3_ref, ol_ref, oh_ref):
  zz = zz_ref[...]
  dot = lambda a, b: jnp.dot(a, b, preferred_element_type=_f32)

  def group(d1_ref, m1_ref, eat_ref, o_ref):
    eat = eat_ref[...]
    t = jnp.maximum(dot(zz, d1_ref[...]) + db1_ref[...], 0.0)
    rec_t = lax.dot_general(d2_ref[...], t, (((0,), (1,)), ((), ())),
                            preferred_element_type=_f32) + db2_ref[...]
    diff = rec_t - eat
    err = jnp.mean(diff * diff, axis=0, keepdims=True)
    m = jnp.maximum(dot(zz, m1_ref[...])
                    + _dgT(eat, m1c_ref[...])
                    + _dgT(err, m1d_ref[...])
                    + mb1_ref[...], 0.0)
    m = jnp.maximum(dot(m, m2_ref[...]) + mb2_ref[...], 0.0)
    o_ref[...] = lax.dot_general(m3_ref[...], m, (((0,), (1,)), ((), ())),
                                 preferred_element_type=_f32) + mb3_ref[...]

  group(d1l_ref, m1l_ref, eatl_ref, ol_ref)
  group(d1h_ref, m1h_ref, eath_ref, oh_ref)


def _head(zz, ea_t, d1, db1, d2, db2, m1, mb1, m2, mb2, m3, mb3):
  nblk = PAIRS // PBLK
  full = lambda shape: pl.BlockSpec(shape, lambda i: tuple(0 for _ in shape))
  zeros64 = jnp.zeros((H, d1.shape[1]), _f32)
  d1_lo = jnp.concatenate([d1, zeros64], axis=0)
  d1_hi = jnp.concatenate([zeros64, d1], axis=0)
  m1ab = m1[:2 * LD]
  z50 = jnp.zeros((H, 50), _f32)
  m1_lo = jnp.concatenate([m1ab, z50], axis=0)
  m1_hi = jnp.concatenate([z50, m1ab], axis=0)
  out = jax.ShapeDtypeStruct((2, PAIRS), _f32)
  return pl.pallas_call(
      _head_body,
      grid=(nblk,),
      in_specs=[
          pl.BlockSpec((PBLK, 2 * H), lambda i: (i, 0)),
          pl.BlockSpec((ED, PBLK), lambda i: (0, i)),
          pl.BlockSpec((ED, PBLK), lambda i: (0, i + PAIRS // PBLK)),
          full((2 * H, H)), full((2 * H, H)), full((1, H)),
          full((H, ED)), full((ED, 1)),
          full((2 * H, 50)), full((2 * H, 50)), full((ED, 50)),
          full((1, 50)), full((1, 50)),
          full((50, 25)), full((1, 25)),
          full((25, 2)), full((2, 1)),
      ],
      out_specs=[pl.BlockSpec((2, PBLK), lambda i: (0, i)),
                 pl.BlockSpec((2, PBLK), lambda i: (0, i))],
      out_shape=[out, out],
  )(zz, ea_t, ea_t,
    d1_lo, d1_hi, db1.reshape(1, H),
    d2, db2.reshape(ED, 1),
    m1_lo, m1_hi, m1[2 * LD:2 * LD + ED],
    m1[2 * LD + ED:].reshape(1, 50), mb1.reshape(1, 50),
    m2, mb2.reshape(1, 25),
    m3, mb3.reshape(2, 1))


@jax.jit
def _run(x, edge_index, edge_attr, params):
  src2d = edge_index[0].reshape(NROW, CHUNK)
  dst2d = edge_index[1].reshape(NROW, CHUNK)
  ea_t = edge_attr.T
  zeros_z = jnp.zeros((CHUNK, H), _f32)

  we, be = params['edge_emb']
  folded = []
  for conv in params['convs']:
    wl, bl = conv['lin_edge']
    folded.append((we @ wl, be @ wl + bl))

  zgather = _make_zgather()
  conv_sc = _make_conv()

  (a1, c1), (a2, c2) = folded
  e2_1, e2_2 = _edge_lin(ea_t, a1, c1, a2, c2)
  e2s = [e2_1, e2_2]

  h = _node_emb(x, params['node_emb'][0], params['node_emb'][1])

  for li, conv in enumerate(params['convs']):
    acc = conv_sc(e2s[li], src2d, dst2d, h, zeros_z)
    scale = (1.0 + conv['eps']).astype(_f32)
    emit_z = li == len(params['convs']) - 1
    h = _node_update(h, acc, scale,
                     conv['nn1'][0], conv['nn1'][1],
                     conv['nn2'][0], conv['nn2'][1],
                     conv['bn_gamma'], conv['bn_beta'],
                     params['enc'][0], params['enc'][1], emit_z)

  z = h
  src_i = edge_index[0]
  dst_i = edge_index[1]
  zidx = jnp.stack([src_i[:PAIRS], dst_i[:PAIRS],
                    src_i[PAIRS:], dst_i[PAIRS:]], axis=1).reshape(-1)
  zz = zgather(z, zidx.reshape(ZNROW, CHUNK)).reshape(PAIRS, 4 * LD)
  out_lo, out_hi = _head(zz, ea_t,
                         params['dec1'][0], params['dec1'][1],
                         params['dec2'][0], params['dec2'][1],
                         params['mlp1'][0], params['mlp1'][1],
                         params['mlp2'][0], params['mlp2'][1],
                         params['mlp3'][0], params['mlp3'][1])
  out_t = jnp.concatenate([out_lo, out_hi], axis=1)
  return out_t.T


def kernel(x, edge_index, edge_attr, params):
  return _run(x, edge_index, edge_attr, params)


# speedup vs baseline: 1.0581x; 1.0004x over previous
import functools

import jax
import jax.numpy as jnp
from jax import lax
from jax.experimental import pallas as pl
from jax.experimental.pallas import tpu as pltpu
from jax.experimental.pallas import tpu_sc as plsc

N = 10000
E = 320000
NF = 128
H = 64
ED = 16
LD = 32

NUM_CORES = 2
NUM_SUBCORES = 16
NUM_TILES = NUM_CORES * NUM_SUBCORES

CHUNK = 128
SROWS = 6
SUPER = SROWS * CHUNK
NROW = E // CHUNK
ROWS_PER_TILE = 78
NSUP = 13
EXTRA = NROW - NUM_TILES * ROWS_PER_TILE
N_PAD = 10016
ZROWS = N_PAD // NUM_SUBCORES
ZHALF = ZROWS // 2
EBLK = 1280
PAIRS = E // 2
PROW = PAIRS // CHUNK
PPT = 39
PEXTRA = PROW - NUM_TILES * PPT
PBLK = EBLK // 2

_f32 = jnp.float32


def _tile_range(wid):
  row0 = ROWS_PER_TILE * wid + jnp.minimum(wid, EXTRA)
  return row0, row0 * CHUNK


ZNROW = 4 * PROW
ZRPT = 156
ZEXTRA = ZNROW - NUM_TILES * ZRPT
ZNSUP = ZRPT // SROWS


def _zgather_body(table, idx_hbm, out_hbm,
                  idx0, idx1, rows0, rows1, tidx, trows,
                  g0, g1, s0, s1, tsem):
  cid = lax.axis_index("c")
  sid = lax.axis_index("s")
  wid = sid * NUM_CORES + cid
  row0 = ZRPT * wid + jnp.minimum(wid, ZEXTRA)
  e0 = row0 * CHUNK
  idx_b = [idx0, idx1]
  row_b = [rows0, rows1]
  gsem = [g0, g1]
  ssem = [s0, s1]
  gd = [None, None]
  sd = [None, None]

  def fire(s):
    b = s & 1
    pltpu.sync_copy(idx_hbm.at[pl.ds(row0 + SROWS * s, SROWS)], idx_b[b])
    gd[b] = [
        pltpu.async_copy(table.at[idx_b[b].at[j]],
                         row_b[b].at[pl.ds(j * CHUNK, CHUNK)], gsem[b])
        for j in range(SROWS)
    ]

  fire(0)
  fire(1)
  for s in range(ZNSUP):
    b = s & 1
    for d in gd[b]:
      d.wait()
    sd[b] = pltpu.async_copy(row_b[b],
                             out_hbm.at[pl.ds(e0 + SUPER * s, SUPER)],
                             ssem[b])
    if s + 2 < ZNSUP:
      sd[b].wait()
      sd[b] = None
      fire(s + 2)
  for b in range(2):
    if sd[b] is not None:
      sd[b].wait()

  @pl.when(wid < ZEXTRA)
  def _():
    pltpu.sync_copy(idx_hbm.at[pl.ds(row0 + ZRPT, 1)], tidx)
    pltpu.async_copy(table.at[tidx.at[0]], trows, tsem).wait()
    pltpu.sync_copy(trows, out_hbm.at[pl.ds(e0 + ZRPT * CHUNK, CHUNK)])


def _make_zgather():
  mesh = plsc.VectorSubcoreMesh(core_axis_name="c", subcore_axis_name="s",
                                num_cores=NUM_CORES,
                                num_subcores=NUM_SUBCORES)
  return pl.kernel(
      _zgather_body,
      out_type=jax.ShapeDtypeStruct((2 * E, LD), _f32),
      mesh=mesh,
      compiler_params=pltpu.CompilerParams(use_tc_tiling_on_sc=False),
      scratch_types=[
          pltpu.VMEM((SROWS, CHUNK), jnp.int32),
          pltpu.VMEM((SROWS, CHUNK), jnp.int32),
          pltpu.VMEM((SUPER, LD), _f32),
          pltpu.VMEM((SUPER, LD), _f32),
          pltpu.VMEM((1, CHUNK), jnp.int32),
          pltpu.VMEM((CHUNK, LD), _f32),
          pltpu.SemaphoreType.DMA,
          pltpu.SemaphoreType.DMA,
          pltpu.SemaphoreType.DMA,
          pltpu.SemaphoreType.DMA,
          pltpu.SemaphoreType.DMA,
      ],
  )


def _conv_body(e2_hbm, src_hbm, dst_hbm, h_hbm, zeros_hbm, out_hbm,
               slo0, slo1, slo2, shi0, shi1, shi2,
               dlo0, dlo1, dlo2, dhi0, dhi1, dhi2,
               eb0, eb1, hlo0, hlo1, hlo2, hhi0, hhi1, hhi2, acc_sh,
               ge0, ge1, gl0, gl1, gl2, gh0, gh1, gh2,
               as0, as1, as2, tsem):
  cid = lax.axis_index("c")
  sid = lax.axis_index("s")
  wid = cid * NUM_SUBCORES + sid
  p0 = PPT * wid + jnp.minimum(wid, PEXTRA)

  pltpu.sync_copy(zeros_hbm, hlo0.at[pl.ds(0, CHUNK)])
  for zz in range(5):
    rows = CHUNK if zz < 4 else ZROWS - 4 * CHUNK
    pltpu.sync_copy(hlo0.at[pl.ds(0, rows)],
                    acc_sh.at[pl.ds(sid * ZROWS + zz * CHUNK, rows)])
  plsc.subcore_barrier()

  slo = [slo0, slo1, slo2]
  shi = [shi0, shi1, shi2]
  dlo = [dlo0, dlo1, dlo2]
  dhi = [dhi0, dhi1, dhi2]
  ebuf = [eb0, eb1]
  hlo = [hlo0, hlo1, hlo2]
  hhi = [hhi0, hhi1, hhi2]
  gesem = [ge0, ge1]
  glsem = [gl0, gl1, gl2]
  ghsem = [gh0, gh1, gh2]
  asem = [as0, as1, as2]
  ed = [None, None]
  gld = [None, None, None]
  ghd = [None, None, None]
  ad = [None, None, None]

  def fire(s):
    b2 = s & 1
    b3 = s % 3
    p = p0 + s
    pltpu.sync_copy(src_hbm.at[pl.ds(p, 1)], slo[b3])
    pltpu.sync_copy(src_hbm.at[pl.ds(PROW + p, 1)], shi[b3])
    pltpu.sync_copy(dst_hbm.at[pl.ds(p, 1)], dlo[b3])
    pltpu.sync_copy(dst_hbm.at[pl.ds(PROW + p, 1)], dhi[b3])
    ed[b2] = pltpu.async_copy(e2_hbm.at[pl.ds(p * CHUNK, CHUNK)], ebuf[b2],
                              gesem[b2])
    gld[b3] = pltpu.async_copy(h_hbm.at[slo[b3].at[0]], hlo[b3], glsem[b3])
    ghd[b3] = pltpu.async_copy(h_hbm.at[shi[b3].at[0]], hhi[b3], ghsem[b3])

  def step(s):
    b2 = s & 1
    b3 = s % 3
    ed[b2].wait()
    gld[b3].wait()
    ghd[b3].wait()

    @plsc.parallel_loop(0, CHUNK, step=1, unroll=1)
    def body(r):
      for c in range(4):
        lo = jnp.maximum(hlo[b3][r, pl.ds(c * 16, 16)]
                         + ebuf[b2][r, pl.ds(c * 16, 16)], 0.0)
        hlo[b3][r, pl.ds(c * 16, 16)] = lo
        hi = jnp.maximum(hhi[b3][r, pl.ds(c * 16, 16)]
                         + ebuf[b2][r, pl.ds(64 + c * 16, 16)], 0.0)
        hhi[b3][r, pl.ds(c * 16, 16)] = hi

    ad[b3] = [
        pltpu.async_copy(hlo[b3], acc_sh.at[dlo[b3].at[0]], asem[b3],
                         add=True),
        pltpu.async_copy(hhi[b3], acc_sh.at[dhi[b3].at[0]], asem[b3],
                         add=True),
    ]

  def drain(s):
    if s < 0:
      return
    b3 = s % 3
    if ad[b3] is not None:
      for d in ad[b3]:
        d.wait()
      ad[b3] = None

  fire(0)
  fire(1)
  for s in range(PPT):
    step(s)
    if s + 2 < PPT:
      drain(s - 1)
      fire(s + 2)
  drain(PPT - 3)
  drain(PPT - 2)
  drain(PPT - 1)

  @pl.when(wid < PEXTRA)
  def _():
    fire(PPT)
    step(PPT)
    drain(PPT)

  plsc.subcore_barrier()
  for hlf in range(2):
    pltpu.sync_copy(acc_sh.at[pl.ds(sid * ZROWS + hlf * ZHALF, ZHALF)],
                    hlo0.at[pl.ds(0, ZHALF)])
    pltpu.sync_copy(hlo0.at[pl.ds(0, ZHALF)],
                    out_hbm.at[cid, pl.ds(sid * ZROWS + hlf * ZHALF, ZHALF)])


def _make_conv():
  mesh = plsc.VectorSubcoreMesh(core_axis_name="c", subcore_axis_name="s",
                                num_cores=NUM_CORES,
                                num_subcores=NUM_SUBCORES)
  return pl.kernel(
      _conv_body,
      out_type=jax.ShapeDtypeStruct((NUM_CORES, N_PAD, H), _f32),
      mesh=mesh,
      compiler_params=pltpu.CompilerParams(use_tc_tiling_on_sc=False),
      scratch_types=(
          [pltpu.VMEM((1, CHUNK), jnp.int32)] * 12
          + [pltpu.VMEM((CHUNK, 2 * H), _f32)] * 2
          + [pltpu.VMEM((CHUNK, H), _f32)] * 6
          + [pltpu.VMEM_SHARED((N_PAD, H), _f32)]
          + [pltpu.SemaphoreType.DMA] * 12
      ),
  )


def _dgT(lhs, rhs):
  return lax.dot_general(lhs, rhs, (((0,), (0,)), ((), ())),
                         preferred_element_type=_f32)


def _node_emb_body(x_ref, w_ref, b_ref, o_ref):
  o_ref[...] = jnp.dot(x_ref[...], w_ref[...],
                       preferred_element_type=_f32) + b_ref[...]


def _node_emb(x, w, b):
  return pl.pallas_call(
      _node_emb_body,
      out_shape=jax.ShapeDtypeStruct((N, H), _f32),
  )(x, w, b.reshape(1, H))


def _edge_lin_body(eat_lo_ref, eat_hi_ref, a1_ref, c1_ref, a2_ref, c2_ref,
                   o1_ref, o2_ref):
  lo = eat_lo_ref[...]
  hi = eat_hi_ref[...]
  o1_ref[...] = jnp.concatenate(
      [_dgT(lo, a1_ref[...]) + c1_ref[...],
       _dgT(hi, a1_ref[...]) + c1_ref[...]], axis=1)
  o2_ref[...] = jnp.concatenate(
      [_dgT(lo, a2_ref[...]) + c2_ref[...],
       _dgT(hi, a2_ref[...]) + c2_ref[...]], axis=1)


def _edge_lin(ea_t, a1, c1, a2, c2):
  eblk = 2 * PBLK
  nblk = PAIRS // eblk
  out = jax.ShapeDtypeStruct((PAIRS, 2 * H), _f32)
  return pl.pallas_call(
      _edge_lin_body,
      grid=(nblk,),
      in_specs=[
          pl.BlockSpec((ED, eblk), lambda i: (0, i)),
          pl.BlockSpec((ED, eblk), lambda i: (0, i + PAIRS // eblk)),
          pl.BlockSpec((ED, H), lambda i: (0, 0)),
          pl.BlockSpec((1, H), lambda i: (0, 0)),
          pl.BlockSpec((ED, H), lambda i: (0, 0)),
          pl.BlockSpec((1, H), lambda i: (0, 0)),
      ],
      out_specs=[pl.BlockSpec((eblk, 2 * H), lambda i: (i, 0)),
                 pl.BlockSpec((eblk, 2 * H), lambda i: (i, 0))],
      out_shape=[out, out],
  )(ea_t, ea_t, a1, c1.reshape(1, H), a2, c2.reshape(1, H))


def _node_update_body(emit_z, h_ref, acc_ref, scale_ref, w1_ref, b1_ref,
                      w2_ref, b2_ref, g_ref, be_ref, ew_ref, eb_ref, o_ref):
  aggr = acc_ref[0, :N, :] + acc_ref[1, :N, :]
  h = h_ref[...]
  t = scale_ref[0, 0] * h + aggr
  t = jnp.maximum(jnp.dot(t, w1_ref[...],
                          preferred_element_type=_f32) + b1_ref[...], 0.0)
  t = jnp.dot(t, w2_ref[...], preferred_element_type=_f32) + b2_ref[...]
  mean = jnp.mean(t, axis=0, keepdims=True)
  var = jnp.mean((t - mean) ** 2, axis=0, keepdims=True)
  t = (t - mean) * lax.rsqrt(var + 1e-5) * g_ref[...] + be_ref[...]
  hn = (h + jnp.maximum(t, 0.0)) * 0.5
  if emit_z:
    o_ref[...] = jnp.dot(hn, ew_ref[...],
                         preferred_element_type=_f32) + eb_ref[...]
  else:
    o_ref[...] = hn


def _node_update(h, acc, scale, w1, b1, w2, b2, gamma, beta, enc_w, enc_b,
                 emit_z):
  out_d = LD if emit_z else H
  return pl.pallas_call(
      functools.partial(_node_update_body, emit_z),
      out_shape=jax.ShapeDtypeStruct((N, out_d), _f32),
  )(h, acc, scale.reshape(1, 1), w1, b1.reshape(1, H), w2, b2.reshape(1, H),
    gamma.reshape(1, H), beta.reshape(1, H), enc_w, enc_b.reshape(1, LD))


def _head_body(zz_ref, eatl_ref, eath_ref, d1l_ref, d1h_ref, db1_ref,
               d2_ref, db2_ref, m1l_ref, m1h_ref, m1c_ref, m1d_ref, mb1_ref,
               m2_ref, mb2_ref, m3_ref, mb3_ref, ol_ref, oh_ref):
  zz = zz_ref[...]
  dot = lambda a, b: jnp.dot(a, b, preferred_element_type=_f32)

  def group(d1_ref, m1_ref, eat_ref, o_ref):
    eat = eat_ref[...]
    t = jnp.maximum(dot(zz, d1_ref[...]) + db1_ref[...], 0.0)
    rec_t = lax.dot_general(d2_ref[...], t, (((0,), (1,)), ((), ())),
                            preferred_element_type=_f32) + db2_ref[...]
    diff = rec_t - eat
    err = jnp.mean(diff * diff, axis=0, keepdims=True)
    m = jnp.maximum(dot(zz, m1_ref[...])
                    + _dgT(eat, m1c_ref[...])
                    + _dgT(err, m1d_ref[...])
                    + mb1_ref[...], 0.0)
    m = jnp.maximum(dot(m, m2_ref[...]) + mb2_ref[...], 0.0)
    o_ref[...] = lax.dot_general(m3_ref[...], m, (((0,), (1,)), ((), ())),
                                 preferred_element_type=_f32) + mb3_ref[...]

  group(d1l_ref, m1l_ref, eatl_ref, ol_ref)
  group(d1h_ref, m1h_ref, eath_ref, oh_ref)


def _head(zz, ea_t, d1, db1, d2, db2, m1, mb1, m2, mb2, m3, mb3):
  nblk = PAIRS // PBLK
  full = lambda shape: pl.BlockSpec(shape, lambda i: tuple(0 for _ in shape))
  zeros64 = jnp.zeros((H, d1.shape[1]), _f32)
  d1_lo = jnp.concatenate([d1, zeros64], axis=0)
  d1_hi = jnp.concatenate([zeros64, d1], axis=0)
  m1ab = m1[:2 * LD]
  z50 = jnp.zeros((H, 50), _f32)
  m1_lo = jnp.concatenate([m1ab, z50], axis=0)
  m1_hi = jnp.concatenate([z50, m1ab], axis=0)
  out = jax.ShapeDtypeStruct((2, PAIRS), _f32)
  return pl.pallas_call(
      _head_body,
      grid=(nblk,),
      in_specs=[
          pl.BlockSpec((PBLK, 2 * H), lambda i: (i, 0)),
          pl.BlockSpec((ED, PBLK), lambda i: (0, i)),
          pl.BlockSpec((ED, PBLK), lambda i: (0, i + PAIRS // PBLK)),
          full((2 * H, H)), full((2 * H, H)), full((1, H)),
          full((H, ED)), full((ED, 1)),
          full((2 * H, 50)), full((2 * H, 50)), full((ED, 50)),
          full((1, 50)), full((1, 50)),
          full((50, 25)), full((1, 25)),
          full((25, 2)), full((2, 1)),
      ],
      out_specs=[pl.BlockSpec((2, PBLK), lambda i: (0, i)),
                 pl.BlockSpec((2, PBLK), lambda i: (0, i))],
      out_shape=[out, out],
  )(zz, ea_t, ea_t,
    d1_lo, d1_hi, db1.reshape(1, H),
    d2, db2.reshape(ED, 1),
    m1_lo, m1_hi, m1[2 * LD:2 * LD + ED],
    m1[2 * LD + ED:].reshape(1, 50), mb1.reshape(1, 50),
    m2, mb2.reshape(1, 25),
    m3, mb3.reshape(2, 1))


@jax.jit
def _run(x, edge_index, edge_attr, params):
  src2d = edge_index[0].reshape(NROW, CHUNK)
  dst2d = edge_index[1].reshape(NROW, CHUNK)
  ea_t = edge_attr.T
  zeros_z = jnp.zeros((CHUNK, H), _f32)

  we, be = params['edge_emb']
  folded = []
  for conv in params['convs']:
    wl, bl = conv['lin_edge']
    folded.append((we @ wl, be @ wl + bl))

  zgather = _make_zgather()
  conv_sc = _make_conv()

  (a1, c1), (a2, c2) = folded
  e2_1, e2_2 = _edge_lin(ea_t, a1, c1, a2, c2)
  e2s = [e2_1, e2_2]

  h = _node_emb(x, params['node_emb'][0], params['node_emb'][1])

  for li, conv in enumerate(params['convs']):
    acc = conv_sc(e2s[li], src2d, dst2d, h, zeros_z)
    scale = (1.0 + conv['eps']).astype(_f32)
    emit_z = li == len(params['convs']) - 1
    h = _node_update(h, acc, scale,
                     conv['nn1'][0], conv['nn1'][1],
                     conv['nn2'][0], conv['nn2'][1],
                     conv['bn_gamma'], conv['bn_beta'],
                     params['enc'][0], params['enc'][1], emit_z)

  z = h
  src_i = edge_index[0]
  dst_i = edge_index[1]
  zidx = jnp.stack([src_i[:PAIRS], dst_i[:PAIRS],
                    src_i[PAIRS:], dst_i[PAIRS:]], axis=1).reshape(-1)
  zz = zgather(z, zidx.reshape(ZNROW, CHUNK)).reshape(PAIRS, 4 * LD)
  out_lo, out_hi = _head(zz, ea_t,
                         params['dec1'][0], params['dec1'][1],
                         params['dec2'][0], params['dec2'][1],
                         params['mlp1'][0], params['mlp1'][1],
                         params['mlp2'][0], params['mlp2'][1],
                         params['mlp3'][0], params['mlp3'][1])
  out_t = jnp.concatenate([out_lo, out_hi], axis=1)
  return out_t.T


def kernel(x, edge_index, edge_attr, params):
  return _run(x, edge_index, edge_attr, params)
